# Initial kernel scaffold; baseline (speedup 1.0000x reference)
import jax
import jax.numpy as jnp
from jax.experimental import pallas as pl


def kernel(pos, edge_index, Ws1, bs1, Wd1, bd1, m1W1, m1b1, m1g, m1be, m1W2, m1b2, Ws2, bs2, Wd2, bd2, m2W1, m2b1, m2g, m2be, m2W2, m2b2, pool_p, fc1W, fc1b, fc2W, fc2b):
    raise NotImplementedError("write your pallas kernel here")


# speedup vs baseline: 5.7554x; 5.7554x over previous
import jax
import jax.numpy as jnp
from jax import lax
from jax.experimental import pallas as pl
from jax.experimental.pallas import tpu as pltpu
from jax.experimental.pallas import tpu_sc as plsc

_N = 50000
_E = 800000
_EPS = 1e-7
_BN = 2048
_NB = 25
_NP = _BN * _NB
_K = 512
_TCAP = 64
_REG = _TCAP + 16
_T3 = 25
_CAP = 2048
_NA = 50048
_RPT = _NA // 16


def _bn_scale(g_ref):
    return g_ref[...] / jnp.sqrt(jnp.float32(1.0 + 1e-5))


def _leaky(x):
    return jnp.where(x >= 0, x, 0.01 * x)


def _tc_a_body(pos_ref, ws_ref, bs_ref, wd_ref, bd_ref, emem_ref, xd_ref):
    p = pos_ref[...]
    xs = jnp.dot(p, ws_ref[...], preferred_element_type=jnp.float32) + bs_ref[...]
    m = jnp.maximum(xs, 0.0) + _EPS
    e = jnp.exp(m)
    emem_ref[:, 0:16] = e
    emem_ref[:, 16:32] = m * e
    xd_ref[...] = jnp.dot(p, wd_ref[...], preferred_element_type=jnp.float32) + bd_ref[...]


def _sc1_body(emem_hbm, src_hbm, dst_hbm, out_hbm, sidx, didx, rows, zbuf, acc, sem):
    c = lax.axis_index("c")
    s = lax.axis_index("s")
    zv = jnp.zeros((16,), jnp.float32)

    def _zb(i, _):
        zbuf[i, 0:16] = zv
        zbuf[i, 16:32] = zv
        return 0
    lax.fori_loop(0, 136, _zb, 0)

    def _za(k, _):
        pltpu.sync_copy(zbuf, acc.at[pl.ds(s * _RPT + k * 136, 136)])
        return 0
    lax.fori_loop(0, 23, _za, 0)
    plsc.subcore_barrier()

    base = c * (_E // 2)

    def _edge_batch(off):
        pltpu.sync_copy(src_hbm.at[pl.ds(off, 128)], sidx)
        pltpu.sync_copy(dst_hbm.at[pl.ds(off, 128)], didx)
        pltpu.async_copy(emem_hbm.at[sidx], rows, sem).wait()
        pltpu.sync_copy(rows, acc.at[didx], add=True)

    def _eb(b, _):
        _edge_batch(base + (b * 16 + s) * 128)
        return 0
    lax.fori_loop(0, 195, _eb, 0)

    @pl.when(s < 5)
    def _():
        _edge_batch(base + (3120 + s) * 128)

    plsc.subcore_barrier()
    for cc in range(2):
        @pl.when(c == cc)
        def _():
            pltpu.sync_copy(acc.at[pl.ds(s * _RPT, _RPT)],
                            out_hbm.at[cc, pl.ds(s * _RPT, _RPT)])


def _tc_b_body(a1_ref, xd1_ref, w1_ref, b1_ref, g_ref, be_ref, w2_ref, b2_ref,
               ws2_ref, bs2_ref, wd2_ref, bd2_ref, emem_ref, xd2_ref):
    den = a1_ref[0, :, 0:16] + a1_ref[1, :, 0:16]
    num = a1_ref[0, :, 16:32] + a1_ref[1, :, 16:32]
    aggr = num / (den + 1e-16)
    h = aggr + xd1_ref[...]
    t = jnp.dot(h, w1_ref[...], preferred_element_type=jnp.float32) + b1_ref[...]
    t = t * _bn_scale(g_ref) + be_ref[...]
    t = jnp.maximum(t, 0.0)
    x1 = jnp.dot(t, w2_ref[...], preferred_element_type=jnp.float32) + b2_ref[...]
    x1 = _leaky(x1)
    m2 = jnp.maximum(
        jnp.dot(x1, ws2_ref[...], preferred_element_type=jnp.float32) + bs2_ref[...],
        0.0) + _EPS
    e2 = jnp.exp(m2)
    me2 = m2 * e2
    for g in range(4):
        emem_ref[g, :, 0:16] = e2[:, g * 16:(g + 1) * 16]
        emem_ref[g, :, 16:32] = me2[:, g * 16:(g + 1) * 16]
    xd2_ref[...] = jnp.dot(x1, wd2_ref[...], preferred_element_type=jnp.float32) + bd2_ref[...]


def _sc2_body(emem_hbm, src_hbm, dst_hbm, out_hbm, sidx, didx, rows, zbuf, acc, sem):
    c = lax.axis_index("c")
    s = lax.axis_index("s")
    zv = jnp.zeros((16,), jnp.float32)

    def _zb(i, _):
        zbuf[i, 0:16] = zv
        zbuf[i, 16:32] = zv
        return 0
    lax.fori_loop(0, 136, _zb, 0)

    def _za(k, _):
        pltpu.sync_copy(zbuf, acc.at[pl.ds(s * _RPT + k * 136, 136)])
        return 0

    def _edge_batch(g, off):
        pltpu.sync_copy(src_hbm.at[pl.ds(off, 128)], sidx)
        pltpu.sync_copy(dst_hbm.at[pl.ds(off, 128)], didx)
        if g:
            goff = jnp.full((16,), g * _N, jnp.int32)

            def _ao(i, _):
                sidx[pl.ds(i * 16, 16)] = sidx[pl.ds(i * 16, 16)] + goff
                return 0
            lax.fori_loop(0, 8, _ao, 0)
        pltpu.async_copy(emem_hbm.at[sidx], rows, sem).wait()
        pltpu.sync_copy(rows, acc.at[didx], add=True)

    for g in range(4):
        @pl.when(c == g // 2)
        def _(g=g):
            lax.fori_loop(0, 23, _za, 0)
            plsc.subcore_barrier()

            def _eb(b, _):
                _edge_batch(g, (b * 16 + s) * 128)
                return 0
            lax.fori_loop(0, 390, _eb, 0)

            @pl.when(s < 10)
            def _():
                _edge_batch(g, (6240 + s) * 128)

            plsc.subcore_barrier()
            pltpu.sync_copy(acc.at[pl.ds(s * _RPT, _RPT)],
                            out_hbm.at[g, pl.ds(s * _RPT, _RPT)])


def _tc_c_body(a2_ref, xd2_ref, w1_ref, b1_ref, g_ref, be_ref, w2_ref, b2_ref,
               pp_ref, x2_ref, sc_ref):
    i = pl.program_id(0)
    parts = []
    for g in range(4):
        den = a2_ref[g, :, 0:16]
        num = a2_ref[g, :, 16:32]
        parts.append(num / (den + 1e-16))
    aggr = jnp.concatenate(parts, axis=1)
    h = aggr + xd2_ref[...]
    t = jnp.dot(h, w1_ref[...], preferred_element_type=jnp.float32) + b1_ref[...]
    t = t * _bn_scale(g_ref) + be_ref[...]
    t = jnp.maximum(t, 0.0)
    x2 = jnp.dot(t, w2_ref[...], preferred_element_type=jnp.float32) + b2_ref[...]
    x2 = _leaky(x2)
    x2_ref[...] = x2
    pp = pp_ref[...]
    pn = jnp.sqrt(jnp.sum(pp * pp)) + 1e-16
    sc = jnp.tanh(jnp.dot(x2, pp, preferred_element_type=jnp.float32) / pn)
    rows = i * _BN + lax.broadcasted_iota(jnp.int32, (_BN, 1), 0)
    sc_ref[...] = jnp.where(rows < _N, sc, -2.0)


def _tc_d_body(sc_ref, pos_ref):
    scv = sc_ref[...]

    def _bis(_, lh):
        lo, hi = lh
        mid = 0.5 * (lo + hi)
        cnt = jnp.sum((scv > mid).astype(jnp.float32))
        ge = cnt >= _K
        return jnp.where(ge, mid, lo), jnp.where(ge, hi, mid)

    lo, _ = lax.fori_loop(0, 48, _bis, (jnp.float32(-1.5), jnp.float32(1.5)))

    sel = scv > lo
    self32 = sel.astype(jnp.float32)
    nself32 = 1.0 - self32
    j128 = lax.broadcasted_iota(jnp.int32, (128, 128), 0)
    l128 = lax.broadcasted_iota(jnp.int32, (128, 128), 1)
    lt_inc = (j128 <= l128).astype(jnp.float32)
    a400 = lax.broadcasted_iota(jnp.int32, (400, 400), 0)
    b400 = lax.broadcasted_iota(jnp.int32, (400, 400), 1)
    blk_lt = ((a400 // 16 == b400 // 16) & (b400 < a400)).astype(jnp.float32)
    blk_all = (a400 // 16 == b400 // 16).astype(jnp.float32)
    ones = jnp.ones((128, 1), jnp.float32)

    def tile_prefix(f):
        rowpfx = jnp.dot(f, lt_inc, preferred_element_type=jnp.float32)
        rsum = jnp.dot(f, ones, preferred_element_type=jnp.float32)
        tilepfx = jnp.dot(blk_lt, rsum, preferred_element_type=jnp.float32)
        return tilepfx + rowpfx - 1.0

    rel = tile_prefix(self32)
    rel_nc = tile_prefix(nself32)
    rsum_s = jnp.dot(self32, ones, preferred_element_type=jnp.float32)
    tile_tot = jnp.dot(blk_all, rsum_s, preferred_element_type=jnp.float32)
    cnt = jnp.minimum(tile_tot, jnp.float32(_TCAP))
    row = lax.broadcasted_iota(jnp.int32, (400, 128), 0)
    col = lax.broadcasted_iota(jnp.int32, (400, 128), 1)
    base = (row // 16) * _REG
    flat = row * 128 + col
    trash = _CAP + (flat % _CAP)
    in_cand = sel & (rel < _TCAP)
    in_fill = (~sel) & (rel_nc < _REG - cnt)
    posf = jnp.where(in_cand, base + rel.astype(jnp.int32),
                     jnp.where(in_fill,
                               base + cnt.astype(jnp.int32) + rel_nc.astype(jnp.int32),
                               trash))
    pos_ref[...] = posf


def _sc3_body(score_hbm, pos_hbm, x2_hbm, cs_hbm, ci_hbm, cx_hbm,
              sbuf, lv, li, posb, ivb, gx, sem):
    c = lax.axis_index("c")
    s = lax.axis_index("s")
    w = s * 2 + c
    npt = _NP // _T3
    npad = _CAP - _T3 * _REG

    @pl.when(w < _T3)
    def _():
        pltpu.sync_copy(score_hbm.at[pl.ds(w * npt, npt)], sbuf)
        pltpu.sync_copy(pos_hbm.at[pl.ds(w * 16, 16)], posb)
        negv = jnp.full((16,), -2.0, jnp.float32)
        zi = jnp.zeros((16,), jnp.int32)
        zv = jnp.zeros((16,), jnp.float32)
        iota = lax.iota(jnp.int32, 16)

        def _fill(i, _):
            ivb[i // 8, pl.ds((i % 8) * 16, 16)] = jnp.minimum(
                w * npt + i * 16 + iota, _N - 1)
            return 0
        lax.fori_loop(0, npt // 16, _fill, 0)

        @pl.when(w == 0)
        def _():
            def _zs(i, _):
                lv[pl.ds(i * 16, 16)] = negv
                li[pl.ds(i * 16, 16)] = zi
                return 0
            lax.fori_loop(0, npad // 16, _zs, 0)

            def _zr(i, _):
                for q in range(4):
                    gx[i, pl.ds(q * 16, 16)] = zv
                return 0
            lax.fori_loop(0, npad, _zr, 0)
            pltpu.sync_copy(lv.at[pl.ds(0, npad)],
                            cs_hbm.at[pl.ds(_T3 * _REG, npad)])
            pltpu.sync_copy(li.at[pl.ds(0, npad)],
                            ci_hbm.at[pl.ds(_T3 * _REG, npad)])
            pltpu.sync_copy(gx.at[pl.ds(0, npad)],
                            cx_hbm.at[pl.ds(_T3 * _REG, npad)])

        for j in range(16):
            pltpu.async_copy(sbuf.at[pl.ds(j * 128, 128)],
                             cs_hbm.at[posb.at[j]], sem).wait()
            pltpu.async_copy(ivb.at[j], ci_hbm.at[posb.at[j]], sem).wait()
            pltpu.async_copy(x2_hbm.at[ivb.at[j]], gx, sem).wait()
            pltpu.async_copy(gx, cx_hbm.at[posb.at[j]], sem).wait()


def _tc_e_body(cs_ref, ci_ref, cx_ref, xp_ref):
    sv = cs_ref[0, :]
    ix = ci_ref[0, :]
    r512 = lax.broadcasted_iota(jnp.int32, (_K, 128), 0).astype(jnp.float32)

    acc = jnp.zeros((_K, 64), jnp.float32)
    for c in range(_CAP // 128):
        sb = sv[c * 128:(c + 1) * 128]
        ib = ix[c * 128:(c + 1) * 128]
        xb = cx_ref[c * 128:(c + 1) * 128, :]
        gt = (sv[:, None] > sb[None, :]).astype(jnp.float32)
        eq = ((sv[:, None] == sb[None, :]) & (ix[:, None] < ib[None, :])).astype(jnp.float32)
        rank = jnp.sum(gt + eq, axis=0)
        oh = (r512 == rank[None, :]).astype(jnp.float32)
        acc = acc + jnp.dot(oh, xb * sb[:, None], preferred_element_type=jnp.float32,
                            precision=jax.lax.Precision.HIGHEST)
    xp_ref[...] = acc


def _tc_f1_body(f_ref, w_ref, b_ref, h_ref):
    k = pl.program_id(1)
    part = jnp.dot(f_ref[...], w_ref[...], preferred_element_type=jnp.float32)

    @pl.when(k == 0)
    def _():
        h_ref[...] = part

    @pl.when(k > 0)
    def _():
        h_ref[...] += part

    @pl.when(k == 15)
    def _():
        hh = h_ref[...] + b_ref[...]
        h_ref[...] = _leaky(hh)


def _tc_f2_body(h_ref, w_ref, b_ref, o_ref):
    o_ref[...] = jnp.dot(h_ref[...], w_ref[...], preferred_element_type=jnp.float32) + b_ref[...]


def _sc_mesh():
    return plsc.VectorSubcoreMesh(core_axis_name="c", subcore_axis_name="s",
                                  num_cores=2, num_subcores=16)


def _sc_edge_scratch():
    f32 = jnp.float32
    return [
        pltpu.VMEM((128,), jnp.int32),
        pltpu.VMEM((128,), jnp.int32),
        pltpu.VMEM((128, 32), f32),
        pltpu.VMEM((136, 32), f32),
        pltpu.VMEM_SHARED((_NA, 32), f32),
        pltpu.SemaphoreType.DMA,
    ]


def _run_sc1(emem1, src, dst):
    return pl.kernel(
        _sc1_body,
        out_type=jax.ShapeDtypeStruct((2, _NA, 32), jnp.float32),
        mesh=_sc_mesh(),
        scratch_types=_sc_edge_scratch(),
        compiler_params=pltpu.CompilerParams(use_tc_tiling_on_sc=False),
    )(emem1, src, dst)


def _run_sc2(emem2f, src, dst):
    return pl.kernel(
        _sc2_body,
        out_type=jax.ShapeDtypeStruct((4, _NA, 32), jnp.float32),
        mesh=_sc_mesh(),
        scratch_types=_sc_edge_scratch(),
        compiler_params=pltpu.CompilerParams(use_tc_tiling_on_sc=False),
    )(emem2f, src, dst)


def _run_sc3(score, posarr, x2):
    f32 = jnp.float32
    return pl.kernel(
        _sc3_body,
        out_type=[jax.ShapeDtypeStruct((2 * _CAP,), f32),
                  jax.ShapeDtypeStruct((2 * _CAP,), jnp.int32),
                  jax.ShapeDtypeStruct((2 * _CAP, 64), f32)],
        mesh=_sc_mesh(),
        scratch_types=[
            pltpu.VMEM((_NP // _T3,), f32),
            pltpu.VMEM((_REG,), f32),
            pltpu.VMEM((_REG,), jnp.int32),
            pltpu.VMEM((16, 128), jnp.int32),
            pltpu.VMEM((16, 128), jnp.int32),
            pltpu.VMEM((128, 64), f32),
            pltpu.SemaphoreType.DMA,
        ],
        compiler_params=pltpu.CompilerParams(use_tc_tiling_on_sc=False),
    )(score, posarr, x2)


def kernel(pos, edge_index, Ws1, bs1, Wd1, bd1, m1W1, m1b1, m1g, m1be, m1W2, m1b2,
           Ws2, bs2, Wd2, bd2, m2W1, m2b1, m2g, m2be, m2W2, m2b2, pool_p,
           fc1W, fc1b, fc2W, fc2b):
    f32 = jnp.float32
    src = edge_index[0]
    dst = edge_index[1]

    emem1, xd1 = pl.pallas_call(
        _tc_a_body,
        grid=(_NB,),
        in_specs=[
            pl.BlockSpec((_BN, 3), lambda i: (i, 0)),
            pl.BlockSpec((3, 16), lambda i: (0, 0)),
            pl.BlockSpec((1, 16), lambda i: (0, 0)),
            pl.BlockSpec((3, 16), lambda i: (0, 0)),
            pl.BlockSpec((1, 16), lambda i: (0, 0)),
        ],
        out_specs=[pl.BlockSpec((_BN, 32), lambda i: (i, 0)),
                   pl.BlockSpec((_BN, 16), lambda i: (i, 0))],
        out_shape=[jax.ShapeDtypeStruct((_N, 32), f32),
                   jax.ShapeDtypeStruct((_N, 16), f32)],
    )(pos, Ws1, bs1.reshape(1, 16), Wd1, bd1.reshape(1, 16))

    acc1 = _run_sc1(emem1, src, dst)

    emem2, xd2 = pl.pallas_call(
        _tc_b_body,
        grid=(_NB,),
        in_specs=[
            pl.BlockSpec((2, _BN, 32), lambda i: (0, i, 0)),
            pl.BlockSpec((_BN, 16), lambda i: (i, 0)),
            pl.BlockSpec((16, 32), lambda i: (0, 0)),
            pl.BlockSpec((1, 32), lambda i: (0, 0)),
            pl.BlockSpec((1, 32), lambda i: (0, 0)),
            pl.BlockSpec((1, 32), lambda i: (0, 0)),
            pl.BlockSpec((32, 16), lambda i: (0, 0)),
            pl.BlockSpec((1, 16), lambda i: (0, 0)),
            pl.BlockSpec((16, 64), lambda i: (0, 0)),
            pl.BlockSpec((1, 64), lambda i: (0, 0)),
            pl.BlockSpec((16, 64), lambda i: (0, 0)),
            pl.BlockSpec((1, 64), lambda i: (0, 0)),
        ],
        out_specs=[pl.BlockSpec((4, _BN, 32), lambda i: (0, i, 0)),
                   pl.BlockSpec((_BN, 64), lambda i: (i, 0))],
        out_shape=[jax.ShapeDtypeStruct((4, _N, 32), f32),
                   jax.ShapeDtypeStruct((_N, 64), f32)],
    )(acc1, xd1, m1W1, m1b1.reshape(1, 32), m1g.reshape(1, 32), m1be.reshape(1, 32),
      m1W2, m1b2.reshape(1, 16), Ws2, bs2.reshape(1, 64), Wd2, bd2.reshape(1, 64))

    acc2 = _run_sc2(emem2.reshape(4 * _N, 32), src, dst)

    x2, score_col = pl.pallas_call(
        _tc_c_body,
        grid=(_NB,),
        in_specs=[
            pl.BlockSpec((4, _BN, 32), lambda i: (0, i, 0)),
            pl.BlockSpec((_BN, 64), lambda i: (i, 0)),
            pl.BlockSpec((64, 128), lambda i: (0, 0)),
            pl.BlockSpec((1, 128), lambda i: (0, 0)),
            pl.BlockSpec((1, 128), lambda i: (0, 0)),
            pl.BlockSpec((1, 128), lambda i: (0, 0)),
            pl.BlockSpec((128, 64), lambda i: (0, 0)),
            pl.BlockSpec((1, 64), lambda i: (0, 0)),
            pl.BlockSpec((64, 1), lambda i: (0, 0)),
        ],
        out_specs=[pl.BlockSpec((_BN, 64), lambda i: (i, 0)),
                   pl.BlockSpec((_BN, 1), lambda i: (i, 0))],
        out_shape=[jax.ShapeDtypeStruct((_N, 64), f32),
                   jax.ShapeDtypeStruct((_NP, 1), f32)],
    )(acc2, xd2, m2W1, m2b1.reshape(1, 128), m2g.reshape(1, 128), m2be.reshape(1, 128),
      m2W2, m2b2.reshape(1, 64), pool_p.reshape(64, 1))

    score = score_col.reshape(_NP)

    posarr = pl.pallas_call(
        _tc_d_body,
        out_shape=jax.ShapeDtypeStruct((_NP // 128, 128), jnp.int32),
    )(score.reshape(_NP // 128, 128))

    cs, ci, cx = _run_sc3(score, posarr, x2)

    xp = pl.pallas_call(
        _tc_e_body,
        out_shape=jax.ShapeDtypeStruct((_K, 64), f32),
    )(cs[:_CAP].reshape(1, _CAP), ci[:_CAP].reshape(1, _CAP), cx[:_CAP])

    flat = xp.T.reshape(1, 64 * _K)

    h = pl.pallas_call(
        _tc_f1_body,
        grid=(8, 16),
        in_specs=[
            pl.BlockSpec((1, 2048), lambda j, k: (0, k)),
            pl.BlockSpec((2048, 512), lambda j, k: (k, j)),
            pl.BlockSpec((1, 512), lambda j, k: (0, j)),
        ],
        out_specs=pl.BlockSpec((1, 512), lambda j, k: (0, j)),
        out_shape=jax.ShapeDtypeStruct((1, 4096), f32),
    )(flat, fc1W, fc1b.reshape(1, 4096))

    out = pl.pallas_call(
        _tc_f2_body,
        out_shape=jax.ShapeDtypeStruct((1, 512), f32),
    )(h, fc2W, fc2b.reshape(1, 512))

    return out.reshape(_K)
